# baseline (speedup 1.0000x reference)
import jax
import jax.numpy as jnp
from jax import lax
from jax.experimental import pallas as pl
from jax.experimental.pallas import tpu as pltpu
from jax.experimental.pallas import tpu_sc as plsc

B = 4096
WL = 5
D = 64
DP = 128
H = 128
T = 50
NI = B * WL

NC, NS = 2, 16
NW = NC * NS
ROWS_W = B // NW

BB = 512


def _prep_body(wi, si, pi, ep, es, wo, so, po, epo, eso):
    zi = jnp.zeros((B, DP - WL), dtype=jnp.int32)
    wo[...] = jnp.concatenate([wi[...], zi], axis=1)
    so[...] = jnp.concatenate([si[...], zi], axis=1)
    po[...] = jnp.concatenate([pi[...], zi], axis=1)
    zt = jnp.zeros((ep.shape[0], DP - D), dtype=jnp.float32)
    epo[...] = jnp.concatenate([ep[...], zt], axis=1)
    eso[...] = jnp.concatenate([es[...], zt], axis=1)


def _prep(words, suffix, prefix, emb_pref, emb_suff):
    n = emb_pref.shape[0]
    return pl.pallas_call(
        _prep_body,
        out_shape=[
            jax.ShapeDtypeStruct((B, DP), jnp.int32),
            jax.ShapeDtypeStruct((B, DP), jnp.int32),
            jax.ShapeDtypeStruct((B, DP), jnp.int32),
            jax.ShapeDtypeStruct((n, DP), jnp.float32),
            jax.ShapeDtypeStruct((n, DP), jnp.float32),
        ],
    )(words, suffix, prefix, emb_pref, emb_suff)


def _sc_gather_body(ew, ep, es, wi, pi, si, ow, op_, os_,
                    ibuf, wv, pv, sv, rw, pb, sb,
                    semi, semw, gp0, gp1, gs0, gs1, ws0, ws1):
    wid = lax.axis_index("s") * NC + lax.axis_index("c")
    b0 = wid * ROWS_W
    rsl2 = pl.ds(b0, ROWS_W)

    for src, dst in ((wi, wv), (pi, pv), (si, sv)):
        pltpu.async_copy(src.at[rsl2, :], ibuf, semi).wait()
        for w in range(WL):
            cols = jnp.full((16,), w, dtype=jnp.int32)
            for j in range(ROWS_W // 16):
                rows = jnp.arange(16, dtype=jnp.int32) + (16 * j)
                dst[w, pl.ds(j * 16, 16)] = plsc.load_gather(ibuf,
                                                             [rows, cols])

    word_cps = []
    for w in range(WL):
        rsl = pl.ds(w * ROWS_W, ROWS_W)
        word_cps.append(pltpu.async_copy(ew.at[wv.at[w]], rw.at[rsl], semw))

    gsem_p, gsem_s, wsem = (gp0, gp1), (gs0, gs1), (ws0, ws1)
    live = {}
    for w in range(WL):
        par = w % 2
        if w >= 2:
            live[(w - 2, "pw")].wait()
            live[(w - 2, "sw")].wait()
        live[(w, "pg")] = pltpu.async_copy(ep.at[pv.at[w]], pb.at[par],
                                           gsem_p[par])
        live[(w, "sg")] = pltpu.async_copy(es.at[sv.at[w]], sb.at[par],
                                           gsem_s[par])
        osl = pl.ds(w * B + b0, ROWS_W)
        live[(w, "pg")].wait()
        live[(w, "pw")] = pltpu.async_copy(pb.at[par], op_.at[osl], wsem[par])
        live[(w, "sg")].wait()
        live[(w, "sw")] = pltpu.async_copy(sb.at[par], os_.at[osl], wsem[par])

    for cp in word_cps:
        cp.wait()
    out_cps = []
    for w in range(WL):
        rsl = pl.ds(w * ROWS_W, ROWS_W)
        osl = pl.ds(w * B + b0, ROWS_W)
        out_cps.append(pltpu.async_copy(rw.at[rsl], ow.at[osl], semw))
    for w in (WL - 2, WL - 1):
        live[(w, "pw")].wait()
        live[(w, "sw")].wait()
    for cp in out_cps:
        cp.wait()


def _sc_gather(emb_word, emb_pref_pad, emb_suff_pad, wip, pip, sip):
    mesh = plsc.VectorSubcoreMesh(core_axis_name="c", subcore_axis_name="s")
    out_t = [
        jax.ShapeDtypeStruct((NI, D), jnp.float32),
        jax.ShapeDtypeStruct((NI, DP), jnp.float32),
        jax.ShapeDtypeStruct((NI, DP), jnp.float32),
    ]
    scratch = [
        pltpu.VMEM((ROWS_W, DP), jnp.int32),
        pltpu.VMEM((WL, ROWS_W), jnp.int32),
        pltpu.VMEM((WL, ROWS_W), jnp.int32),
        pltpu.VMEM((WL, ROWS_W), jnp.int32),
        pltpu.VMEM((WL * ROWS_W, D), jnp.float32),
        pltpu.VMEM((2, ROWS_W, DP), jnp.float32),
        pltpu.VMEM((2, ROWS_W, DP), jnp.float32),
    ] + [pltpu.SemaphoreType.DMA] * 8
    k = pl.kernel(_sc_gather_body, out_type=out_t, mesh=mesh,
                  scratch_types=scratch,
                  compiler_params=pltpu.CompilerParams(
                      use_tc_tiling_on_sc=False,
                      needs_layout_passes=False))
    return k(emb_word, emb_pref_pad, emb_suff_pad, wip, pip, sip)


def _mlp_body(hw, hp, hs, w1, b1, w2, b2, out):
    acc = jnp.zeros((BB, H), dtype=jnp.float32) + b1[...]
    for w in range(WL):
        avg = (hw[w] + hp[w][:, :D] + hs[w][:, :D]) * (1.0 / 3.0)
        acc = acc + jnp.dot(avg, w1[w * D:(w + 1) * D, :],
                            preferred_element_type=jnp.float32)
    h2 = jnp.tanh(acc)
    o = jnp.dot(h2, w2[...], preferred_element_type=jnp.float32) + b2[...]
    m = jnp.max(o, axis=1, keepdims=True)
    s = o - m
    lse = jnp.log(jnp.sum(jnp.exp(s), axis=1, keepdims=True))
    out[...] = s - lse


def _mlp(hw, hp, hs, W1, b1, W2, b2, *, interpret=False):
    return pl.pallas_call(
        _mlp_body,
        grid=(B // BB,),
        in_specs=[
            pl.BlockSpec((WL, BB, D), lambda i: (0, i, 0)),
            pl.BlockSpec((WL, BB, DP), lambda i: (0, i, 0)),
            pl.BlockSpec((WL, BB, DP), lambda i: (0, i, 0)),
            pl.BlockSpec((WL * D, H), lambda i: (0, 0)),
            pl.BlockSpec((1, H), lambda i: (0, 0)),
            pl.BlockSpec((H, T), lambda i: (0, 0)),
            pl.BlockSpec((1, T), lambda i: (0, 0)),
        ],
        out_specs=pl.BlockSpec((BB, T), lambda i: (i, 0)),
        out_shape=jax.ShapeDtypeStruct((B, T), jnp.float32),
        interpret=interpret,
    )(hw, hp, hs, W1, b1.reshape(1, H), W2, b2.reshape(1, T))


def kernel(words, suffix, prefix, emb_word, emb_pref, emb_suff, W1, b1, W2, b2):
    wip, sip, pip, ppad, spad = _prep(words, suffix, prefix,
                                      emb_pref, emb_suff)
    hw, hp, hs = _sc_gather(emb_word, ppad, spad, wip, pip, sip)
    hw = hw.reshape(WL, B, D)
    hp = hp.reshape(WL, B, DP)
    hs = hs.reshape(WL, B, DP)
    return _mlp(hw, hp, hs, W1, b1, W2, b2)


# speedup vs baseline: 1.0426x; 1.0426x over previous
import jax
import jax.numpy as jnp
from jax import lax
from jax.experimental import pallas as pl
from jax.experimental.pallas import tpu as pltpu
from jax.experimental.pallas import tpu_sc as plsc

B = 4096
WL = 5
D = 64
DP = 128
H = 128
T = 50
NI = B * WL

NC, NS = 2, 16
NW = NC * NS
ROWS_W = B // NW

BB = 512


def _prep_body(ep, es, epo, eso):
    zt = jnp.zeros((ep.shape[0], DP - D), dtype=jnp.float32)
    epo[...] = jnp.concatenate([ep[...], zt], axis=1)
    eso[...] = jnp.concatenate([es[...], zt], axis=1)


def _prep(emb_pref, emb_suff):
    n = emb_pref.shape[0]
    return pl.pallas_call(
        _prep_body,
        out_shape=[
            jax.ShapeDtypeStruct((n, DP), jnp.float32),
            jax.ShapeDtypeStruct((n, DP), jnp.float32),
        ],
    )(emb_pref, emb_suff)


def _sc_gather_body(ew, ep, es, wi, pi, si, ow, op_, os_,
                    wv2, pv2, sv2, wv, pv, sv, rw, pb, sb,
                    semi, semw, gp0, gp1, gs0, gs1, ws0, ws1):
    wid = lax.axis_index("s") * NC + lax.axis_index("c")
    b0 = wid * ROWS_W
    rsl2 = pl.ds(b0, ROWS_W)

    idx_cps = [
        pltpu.async_copy(wi.at[rsl2, :], wv2, semi),
        pltpu.async_copy(pi.at[rsl2, :], pv2, semi),
        pltpu.async_copy(si.at[rsl2, :], sv2, semi),
    ]
    for cp in idx_cps:
        cp.wait()
    for src, dst in ((wv2, wv), (pv2, pv), (sv2, sv)):
        for w in range(WL):
            cols = jnp.full((16,), w, dtype=jnp.int32)
            for j in range(ROWS_W // 16):
                rows = jnp.arange(16, dtype=jnp.int32) + (16 * j)
                dst[w, pl.ds(j * 16, 16)] = plsc.load_gather(src,
                                                             [rows, cols])

    word_cps = []
    for w in range(WL):
        rsl = pl.ds(w * ROWS_W, ROWS_W)
        word_cps.append(pltpu.async_copy(ew.at[wv.at[w]], rw.at[rsl], semw))

    gsem_p, gsem_s, wsem = (gp0, gp1), (gs0, gs1), (ws0, ws1)
    live = {}
    for w in range(WL):
        par = w % 2
        if w >= 2:
            live[(w - 2, "pw")].wait()
            live[(w - 2, "sw")].wait()
        live[(w, "pg")] = pltpu.async_copy(ep.at[pv.at[w]], pb.at[par],
                                           gsem_p[par])
        live[(w, "sg")] = pltpu.async_copy(es.at[sv.at[w]], sb.at[par],
                                           gsem_s[par])
        osl = pl.ds(w * B + b0, ROWS_W)
        live[(w, "pg")].wait()
        live[(w, "pw")] = pltpu.async_copy(pb.at[par], op_.at[osl], wsem[par])
        live[(w, "sg")].wait()
        live[(w, "sw")] = pltpu.async_copy(sb.at[par], os_.at[osl], wsem[par])

    for cp in word_cps:
        cp.wait()
    out_cps = []
    for w in range(WL):
        rsl = pl.ds(w * ROWS_W, ROWS_W)
        osl = pl.ds(w * B + b0, ROWS_W)
        out_cps.append(pltpu.async_copy(rw.at[rsl], ow.at[osl], semw))
    for w in (WL - 2, WL - 1):
        live[(w, "pw")].wait()
        live[(w, "sw")].wait()
    for cp in out_cps:
        cp.wait()


def _sc_gather(emb_word, emb_pref_pad, emb_suff_pad, wip, pip, sip):
    mesh = plsc.VectorSubcoreMesh(core_axis_name="c", subcore_axis_name="s")
    out_t = [
        jax.ShapeDtypeStruct((NI, D), jnp.float32),
        jax.ShapeDtypeStruct((NI, DP), jnp.float32),
        jax.ShapeDtypeStruct((NI, DP), jnp.float32),
    ]
    scratch = [
        pltpu.VMEM((ROWS_W, WL), jnp.int32),
        pltpu.VMEM((ROWS_W, WL), jnp.int32),
        pltpu.VMEM((ROWS_W, WL), jnp.int32),
        pltpu.VMEM((WL, ROWS_W), jnp.int32),
        pltpu.VMEM((WL, ROWS_W), jnp.int32),
        pltpu.VMEM((WL, ROWS_W), jnp.int32),
        pltpu.VMEM((WL * ROWS_W, D), jnp.float32),
        pltpu.VMEM((2, ROWS_W, DP), jnp.float32),
        pltpu.VMEM((2, ROWS_W, DP), jnp.float32),
    ] + [pltpu.SemaphoreType.DMA] * 8
    k = pl.kernel(_sc_gather_body, out_type=out_t, mesh=mesh,
                  scratch_types=scratch,
                  compiler_params=pltpu.CompilerParams(
                      use_tc_tiling_on_sc=False,
                      needs_layout_passes=False))
    return k(emb_word, emb_pref_pad, emb_suff_pad, wip, pip, sip)


def _mlp_body(hw, hp, hs, w1, b1, w2, b2, out):
    acc = jnp.zeros((BB, H), dtype=jnp.float32) + b1[...]
    for w in range(WL):
        avg = (hw[w] + hp[w][:, :D] + hs[w][:, :D]) * (1.0 / 3.0)
        acc = acc + jnp.dot(avg, w1[w * D:(w + 1) * D, :],
                            preferred_element_type=jnp.float32)
    h2 = jnp.tanh(acc)
    o = jnp.dot(h2, w2[...], preferred_element_type=jnp.float32) + b2[...]
    m = jnp.max(o, axis=1, keepdims=True)
    s = o - m
    lse = jnp.log(jnp.sum(jnp.exp(s), axis=1, keepdims=True))
    out[...] = s - lse


def _mlp(hw, hp, hs, W1, b1, W2, b2, *, interpret=False):
    return pl.pallas_call(
        _mlp_body,
        grid=(B // BB,),
        in_specs=[
            pl.BlockSpec((WL, BB, D), lambda i: (0, i, 0)),
            pl.BlockSpec((WL, BB, DP), lambda i: (0, i, 0)),
            pl.BlockSpec((WL, BB, DP), lambda i: (0, i, 0)),
            pl.BlockSpec((WL * D, H), lambda i: (0, 0)),
            pl.BlockSpec((1, H), lambda i: (0, 0)),
            pl.BlockSpec((H, T), lambda i: (0, 0)),
            pl.BlockSpec((1, T), lambda i: (0, 0)),
        ],
        out_specs=pl.BlockSpec((BB, T), lambda i: (i, 0)),
        out_shape=jax.ShapeDtypeStruct((B, T), jnp.float32),
        interpret=interpret,
    )(hw, hp, hs, W1, b1.reshape(1, H), W2, b2.reshape(1, T))


def kernel(words, suffix, prefix, emb_word, emb_pref, emb_suff, W1, b1, W2, b2):
    ppad, spad = _prep(emb_pref, emb_suff)
    hw, hp, hs = _sc_gather(emb_word, ppad, spad, words, prefix, suffix)
    hw = hw.reshape(WL, B, D)
    hp = hp.reshape(WL, B, DP)
    hs = hs.reshape(WL, B, DP)
    return _mlp(hw, hp, hs, W1, b1, W2, b2)


# speedup vs baseline: 1.0570x; 1.0138x over previous
import jax
import jax.numpy as jnp
from jax import lax
from jax.experimental import pallas as pl
from jax.experimental.pallas import tpu as pltpu
from jax.experimental.pallas import tpu_sc as plsc

B = 4096
WL = 5
D = 64
DP = 128
H = 128
T = 50
NI = B * WL

NC, NS = 2, 16
NW = NC * NS
ROWS_W = B // NW

BB = 1024


def _prep_body(ep, es, epo, eso):
    zt = jnp.zeros((ep.shape[0], DP - D), dtype=jnp.float32)
    epo[...] = jnp.concatenate([ep[...], zt], axis=1)
    eso[...] = jnp.concatenate([es[...], zt], axis=1)


def _prep(emb_pref, emb_suff):
    n = emb_pref.shape[0]
    return pl.pallas_call(
        _prep_body,
        out_shape=[
            jax.ShapeDtypeStruct((n, DP), jnp.float32),
            jax.ShapeDtypeStruct((n, DP), jnp.float32),
        ],
    )(emb_pref, emb_suff)


def _sc_gather_body(ew, ep, es, wi, pi, si, ow, op_, os_,
                    wv2, pv2, sv2, wv, pv, sv, rw, pb, sb,
                    semi, semw, gp0, gp1, gs0, gs1, ws0, ws1):
    wid = lax.axis_index("s") * NC + lax.axis_index("c")
    b0 = wid * ROWS_W
    rsl2 = pl.ds(b0, ROWS_W)

    idx_cps = [
        pltpu.async_copy(wi.at[rsl2, :], wv2, semi),
        pltpu.async_copy(pi.at[rsl2, :], pv2, semi),
        pltpu.async_copy(si.at[rsl2, :], sv2, semi),
    ]
    for cp in idx_cps:
        cp.wait()
    for src, dst in ((wv2, wv), (pv2, pv), (sv2, sv)):
        for w in range(WL):
            cols = jnp.full((16,), w, dtype=jnp.int32)
            for j in range(ROWS_W // 16):
                rows = jnp.arange(16, dtype=jnp.int32) + (16 * j)
                dst[w, pl.ds(j * 16, 16)] = plsc.load_gather(src,
                                                             [rows, cols])

    word_cps = []
    for w in range(WL):
        rsl = pl.ds(w * ROWS_W, ROWS_W)
        word_cps.append(pltpu.async_copy(ew.at[wv.at[w]], rw.at[rsl], semw))

    gsem_p, gsem_s, wsem = (gp0, gp1), (gs0, gs1), (ws0, ws1)
    live = {}
    for w in range(WL):
        par = w % 2
        if w >= 2:
            live[(w - 2, "pw")].wait()
            live[(w - 2, "sw")].wait()
        live[(w, "pg")] = pltpu.async_copy(ep.at[pv.at[w]], pb.at[par],
                                           gsem_p[par])
        live[(w, "sg")] = pltpu.async_copy(es.at[sv.at[w]], sb.at[par],
                                           gsem_s[par])
        osl = pl.ds(w * B + b0, ROWS_W)
        live[(w, "pg")].wait()
        live[(w, "pw")] = pltpu.async_copy(pb.at[par], op_.at[osl], wsem[par])
        live[(w, "sg")].wait()
        live[(w, "sw")] = pltpu.async_copy(sb.at[par], os_.at[osl], wsem[par])

    for cp in word_cps:
        cp.wait()
    out_cps = []
    for w in range(WL):
        rsl = pl.ds(w * ROWS_W, ROWS_W)
        osl = pl.ds(w * B + b0, ROWS_W)
        out_cps.append(pltpu.async_copy(rw.at[rsl], ow.at[osl], semw))
    for w in (WL - 2, WL - 1):
        live[(w, "pw")].wait()
        live[(w, "sw")].wait()
    for cp in out_cps:
        cp.wait()


def _sc_gather(emb_word, emb_pref_pad, emb_suff_pad, wip, pip, sip):
    mesh = plsc.VectorSubcoreMesh(core_axis_name="c", subcore_axis_name="s")
    out_t = [
        jax.ShapeDtypeStruct((NI, D), jnp.float32),
        jax.ShapeDtypeStruct((NI, DP), jnp.float32),
        jax.ShapeDtypeStruct((NI, DP), jnp.float32),
    ]
    scratch = [
        pltpu.VMEM((ROWS_W, WL), jnp.int32),
        pltpu.VMEM((ROWS_W, WL), jnp.int32),
        pltpu.VMEM((ROWS_W, WL), jnp.int32),
        pltpu.VMEM((WL, ROWS_W), jnp.int32),
        pltpu.VMEM((WL, ROWS_W), jnp.int32),
        pltpu.VMEM((WL, ROWS_W), jnp.int32),
        pltpu.VMEM((WL * ROWS_W, D), jnp.float32),
        pltpu.VMEM((2, ROWS_W, DP), jnp.float32),
        pltpu.VMEM((2, ROWS_W, DP), jnp.float32),
    ] + [pltpu.SemaphoreType.DMA] * 8
    k = pl.kernel(_sc_gather_body, out_type=out_t, mesh=mesh,
                  scratch_types=scratch,
                  compiler_params=pltpu.CompilerParams(
                      use_tc_tiling_on_sc=False,
                      needs_layout_passes=False))
    return k(emb_word, emb_pref_pad, emb_suff_pad, wip, pip, sip)


def _mlp_body(hw, hp, hs, w1, b1, w2, b2, out):
    acc = jnp.zeros((BB, H), dtype=jnp.float32) + b1[...]
    for w in range(WL):
        avg = (hw[w] + hp[w][:, :D] + hs[w][:, :D]) * (1.0 / 3.0)
        acc = acc + jnp.dot(avg, w1[w * D:(w + 1) * D, :],
                            preferred_element_type=jnp.float32)
    h2 = jnp.tanh(acc)
    o = jnp.dot(h2, w2[...], preferred_element_type=jnp.float32) + b2[...]
    m = jnp.max(o, axis=1, keepdims=True)
    s = o - m
    lse = jnp.log(jnp.sum(jnp.exp(s), axis=1, keepdims=True))
    out[...] = s - lse


def _mlp(hw, hp, hs, W1, b1, W2, b2, *, interpret=False):
    return pl.pallas_call(
        _mlp_body,
        grid=(B // BB,),
        in_specs=[
            pl.BlockSpec((WL, BB, D), lambda i: (0, i, 0)),
            pl.BlockSpec((WL, BB, DP), lambda i: (0, i, 0)),
            pl.BlockSpec((WL, BB, DP), lambda i: (0, i, 0)),
            pl.BlockSpec((WL * D, H), lambda i: (0, 0)),
            pl.BlockSpec((1, H), lambda i: (0, 0)),
            pl.BlockSpec((H, T), lambda i: (0, 0)),
            pl.BlockSpec((1, T), lambda i: (0, 0)),
        ],
        out_specs=pl.BlockSpec((BB, T), lambda i: (i, 0)),
        out_shape=jax.ShapeDtypeStruct((B, T), jnp.float32),
        interpret=interpret,
    )(hw, hp, hs, W1, b1.reshape(1, H), W2, b2.reshape(1, T))


def kernel(words, suffix, prefix, emb_word, emb_pref, emb_suff, W1, b1, W2, b2):
    ppad, spad = _prep(emb_pref, emb_suff)
    hw, hp, hs = _sc_gather(emb_word, ppad, spad, words, prefix, suffix)
    hw = hw.reshape(WL, B, D)
    hp = hp.reshape(WL, B, DP)
    hs = hs.reshape(WL, B, DP)
    return _mlp(hw, hp, hs, W1, b1, W2, b2)
